# baseline (speedup 1.0000x reference)
import functools

import jax
import jax.numpy as jnp
from jax import lax
from jax.experimental import pallas as pl
from jax.experimental.pallas import tpu as pltpu
from jax.experimental.pallas import tpu_sc as plsc

NC = 2
NS = 16
EK = 200
RW = 400
PB = 400


def _zero_rows(rows_v, nrows):
    def zrow(i, carry):
        rows_v[0, i, :] = jnp.zeros((16,), jnp.float32)
        return carry
    lax.fori_loop(0, nrows, zrow, 0)


def _zero_agg(agg, rows_v, rbase, rpt):
    def zcp(j, carry):
        pltpu.sync_copy(rows_v.at[0, pl.ds(0, RW)],
                        agg.at[pl.ds(rbase + j * RW, RW)])
        return carry
    lax.fori_loop(0, rpt // RW, zcp, 0)


RING = 3


def _agg_edges(table, src_a, dst_a, agg, src_v, dst_v, rows_v,
               lsem, gsem, ssems, ebase, nch):
    def start_load(j, b):
        off = ebase + j * EK
        pltpu.async_copy(src_a.at[pl.ds(off, EK)], src_v.at[b], lsem)
        pltpu.async_copy(dst_a.at[pl.ds(off, EK)], dst_v.at[b], lsem)

    def wait_load(b):
        pltpu.make_async_copy(src_a.at[pl.ds(0, EK)], src_v.at[b],
                              lsem).wait()
        pltpu.make_async_copy(dst_a.at[pl.ds(0, EK)], dst_v.at[b],
                              lsem).wait()

    def wait_scatter(b):
        pltpu.make_async_copy(rows_v.at[b], agg.at[dst_v.at[b]],
                              ssems[b]).wait()

    start_load(0, 0)
    n_outer = (nch + RING - 1) // RING

    def outer(t, carry):
        j0 = t * RING
        for b in range(RING):
            j = j0 + b

            @pl.when(j < nch)
            def _():
                wait_load(b)

                @pl.when(j >= 2)
                def _():
                    wait_scatter((b + 1) % RING)

                @pl.when(j + 1 < nch)
                def _():
                    start_load(j + 1, (b + 1) % RING)

                pltpu.async_copy(table.at[src_v.at[b]], rows_v.at[b],
                                 gsem).wait()
                pltpu.async_copy(rows_v.at[b], agg.at[dst_v.at[b]],
                                 ssems[b], add=True)
        return carry

    lax.fori_loop(0, n_outer, outer, 0)
    for k in (2, 1):
        if nch - k >= 0:
            wait_scatter((nch - k) % RING)


def _write_out(agg, rows_v, out, c, rbase, rpt):
    def wcp(j, carry):
        r0 = rbase + j * RW
        pltpu.sync_copy(agg.at[pl.ds(r0, RW)], rows_v.at[0, pl.ds(0, RW)])
        pltpu.sync_copy(rows_v.at[0, pl.ds(0, RW)], out.at[c, pl.ds(r0, RW)])
        return carry
    lax.fori_loop(0, rpt // RW, wcp, 0)


def _sc_agg_layer1(n, e):
    mesh = plsc.VectorSubcoreMesh(core_axis_name="c", subcore_axis_name="s")

    @functools.partial(
        pl.kernel,
        out_type=jax.ShapeDtypeStruct((NC, n, 16), jnp.float32),
        mesh=mesh,
        scratch_types=[
            pltpu.VMEM_SHARED((n, 16), jnp.float32),
            pltpu.VMEM((RING, EK), jnp.int32),
            pltpu.VMEM((RING, EK), jnp.int32),
            pltpu.VMEM((RING, EK, 16), jnp.float32),
            pltpu.SemaphoreType.DMA,
            pltpu.SemaphoreType.DMA,
            pltpu.SemaphoreType.DMA,
            pltpu.SemaphoreType.DMA,
            pltpu.SemaphoreType.DMA,
        ],
        compiler_params=pltpu.CompilerParams(use_tc_tiling_on_sc=False),
    )
    def k(table, src_a, dst_a, out, agg, src_v, dst_v, rows_v,
          lsem, gsem, ssem0, ssem1, ssem2):
        c = lax.axis_index("c")
        s = lax.axis_index("s")
        rpt = n // NS
        rbase = s * rpt
        _zero_rows(rows_v, RW)
        _zero_agg(agg, rows_v, rbase, rpt)
        plsc.subcore_barrier()
        wid = s * NC + c
        ept = e // (NC * NS)
        _agg_edges(table, src_a, dst_a, agg, src_v, dst_v, rows_v,
                   lsem, gsem, (ssem0, ssem1, ssem2),
                   wid * ept, ept // EK)
        plsc.subcore_barrier()
        _write_out(agg, rows_v, out, c, rbase, rpt)

    return k


def _sc_agg_layer2(n, e):
    mesh = plsc.VectorSubcoreMesh(core_axis_name="c", subcore_axis_name="s")

    @functools.partial(
        pl.kernel,
        out_type=jax.ShapeDtypeStruct((NC, n, 16), jnp.float32),
        mesh=mesh,
        scratch_types=[
            pltpu.VMEM_SHARED((n, 16), jnp.float32),
            pltpu.VMEM((RING, EK), jnp.int32),
            pltpu.VMEM((RING, EK), jnp.int32),
            pltpu.VMEM((RING, EK, 16), jnp.float32),
            pltpu.SemaphoreType.DMA,
            pltpu.SemaphoreType.DMA,
            pltpu.SemaphoreType.DMA,
            pltpu.SemaphoreType.DMA,
            pltpu.SemaphoreType.DMA,
        ],
        compiler_params=pltpu.CompilerParams(use_tc_tiling_on_sc=False),
    )
    def k(t_lo, t_hi, src_a, dst_a, out, agg, src_v, dst_v, rows_v,
          lsem, gsem, ssem0, ssem1, ssem2):
        c = lax.axis_index("c")
        s = lax.axis_index("s")
        rpt = n // NS
        rbase = s * rpt
        _zero_rows(rows_v, RW)
        _zero_agg(agg, rows_v, rbase, rpt)
        plsc.subcore_barrier()
        ept = e // NS
        ebase = s * ept
        nch = ept // EK
        ssems = (ssem0, ssem1, ssem2)

        @pl.when(c == 0)
        def _():
            _agg_edges(t_lo, src_a, dst_a, agg, src_v, dst_v, rows_v,
                       lsem, gsem, ssems, ebase, nch)

        @pl.when(c == 1)
        def _():
            _agg_edges(t_hi, src_a, dst_a, agg, src_v, dst_v, rows_v,
                       lsem, gsem, ssems, ebase, nch)

        plsc.subcore_barrier()
        _write_out(agg, rows_v, out, c, rbase, rpt)

    return k


def _dense1(p_p, xa_p, w1l_lo, w1l_hi, w1r_lo, w1r_hi, b1lo, b1hi, b5):
    m = p_p.shape[1]

    def body(p_ref, xa_ref, a_ref, b_ref, c_ref, d_ref, e_ref, f_ref,
             b5_ref, lo_ref, hi_ref):
        dot = functools.partial(jnp.dot, preferred_element_type=jnp.float32)
        psum = p_ref[0] + p_ref[1]
        degb = jnp.maximum(dot(psum, b5_ref[...]), 1.0)
        mean = psum / degb
        xa = xa_ref[...]
        lo_ref[...] = jnp.maximum(
            dot(mean, a_ref[...]) + dot(xa, c_ref[...]) + e_ref[...], 0.0)
        hi_ref[...] = jnp.maximum(
            dot(mean, b_ref[...]) + dot(xa, d_ref[...]) + f_ref[...], 0.0)

    full = pl.BlockSpec((128, 128), lambda i: (0, 0))
    row = pl.BlockSpec((1, 128), lambda i: (0, 0))
    blk = pl.BlockSpec((PB, 128), lambda i: (i, 0))
    return pl.pallas_call(
        body,
        grid=(m // PB,),
        in_specs=[pl.BlockSpec((NC, PB, 128), lambda i: (0, i, 0)), blk,
                  full, full, full, full, row, row, full],
        out_specs=[blk, blk],
        out_shape=[jax.ShapeDtypeStruct((m, 128), jnp.float32)] * 2,
    )(p_p, xa_p, w1l_lo, w1l_hi, w1r_lo, w1r_hi, b1lo, b1hi, b5)


def _dense_r2(h1lo_p, h1hi_p, p_p, w2r_ll, w2r_hl, w2r_lh, w2r_hh,
              b2lo, b2hi, b5):
    m = p_p.shape[1]

    def body(lo_ref, hi_ref, p_ref, g_ref, h_ref, i_ref, j_ref,
             k_ref, l_ref, b5_ref, r2lo_ref, r2hi_ref, deg_ref):
        dot = functools.partial(jnp.dot, preferred_element_type=jnp.float32)
        h1lo = lo_ref[...]
        h1hi = hi_ref[...]
        r2lo_ref[...] = dot(h1lo, g_ref[...]) + dot(h1hi, h_ref[...]) \
            + k_ref[...]
        r2hi_ref[...] = dot(h1lo, i_ref[...]) + dot(h1hi, j_ref[...]) \
            + l_ref[...]
        psum = p_ref[0] + p_ref[1]
        deg_ref[...] = jnp.maximum(dot(psum, b5_ref[...]), 1.0)

    full = pl.BlockSpec((128, 128), lambda i: (0, 0))
    row = pl.BlockSpec((1, 128), lambda i: (0, 0))
    blk = pl.BlockSpec((PB, 128), lambda i: (i, 0))
    return pl.pallas_call(
        body,
        grid=(m // PB,),
        in_specs=[blk, blk, pl.BlockSpec((NC, PB, 128), lambda i: (0, i, 0)),
                  full, full, full, full, row, row, full],
        out_specs=[blk, blk, blk],
        out_shape=[jax.ShapeDtypeStruct((m, 128), jnp.float32)] * 3,
    )(h1lo_p, h1hi_p, p_p, w2r_ll, w2r_hl, w2r_lh, w2r_hh, b2lo, b2hi, b5)


PB2 = 128


def _dense2(q_p, r2lo_p, r2hi_p, degb_p, w2l_ll, w2l_hl, w2l_lh, w2l_hh,
            wo_lo, wo_hi, bo, sh1, sh2, sh3, sh4, bmax, s5, g):
    m = q_p.shape[1]

    def body(q_ref, r2lo_ref, r2hi_ref, deg_ref, a_ref, b_ref, c_ref, d_ref,
             e_ref, f_ref, bo_ref, s1_ref, s2_ref, s3_ref, s4_ref,
             bm_ref, s5_ref, g_ref, out_ref):
        dot = functools.partial(jnp.dot, preferred_element_type=jnp.float32)
        degb = deg_ref[...]
        mlo = q_ref[0] / degb
        mhi = q_ref[1] / degb
        h2lo = jnp.maximum(
            dot(mlo, a_ref[...]) + dot(mhi, b_ref[...]) + r2lo_ref[...], 0.0)
        h2hi = jnp.maximum(
            dot(mlo, c_ref[...]) + dot(mhi, d_ref[...]) + r2hi_ref[...], 0.0)
        lg = dot(h2lo, e_ref[...]) + dot(h2hi, f_ref[...]) + bo_ref[...]
        mx = jnp.maximum(lg, dot(lg, s1_ref[...]))
        mx = jnp.maximum(mx, dot(lg, s2_ref[...]))
        mx = jnp.maximum(mx, dot(lg, s3_ref[...]))
        mx = jnp.maximum(mx, dot(lg, s4_ref[...]))
        mb = dot(mx, bm_ref[...])
        ez = jnp.exp(lg - mb)
        sb = dot(ez, s5_ref[...])
        probs = ez / sb
        pt = jnp.transpose(probs)
        acc = dot(pt[0:5, :], g_ref[0])
        for a in range(1, 8):
            acc = acc + dot(pt[16 * a:16 * a + 5, :], g_ref[a])
        out_ref[...] = acc

    full = pl.BlockSpec((128, 128), lambda i: (0, 0))
    row = pl.BlockSpec((1, 128), lambda i: (0, 0))
    blk = pl.BlockSpec((PB2, 128), lambda i: (i, 0))
    return pl.pallas_call(
        body,
        grid=(m // PB2,),
        in_specs=[pl.BlockSpec((NC, PB2, 128), lambda i: (0, i, 0)),
                  blk, blk, blk,
                  full, full, full, full, full, full, row,
                  full, full, full, full, full, full,
                  pl.BlockSpec((8, PB2, 8 * PB2), lambda i: (0, 0, 0))],
        out_specs=pl.BlockSpec((5, 8 * PB2), lambda i: (0, i)),
        out_shape=jax.ShapeDtypeStruct((5, 8 * m), jnp.float32),
    )(q_p, r2lo_p, r2hi_p, degb_p, w2l_ll, w2l_hl, w2l_lh, w2l_hh,
      wo_lo, wo_hi, bo, sh1, sh2, sh3, sh4, bmax, s5, g)


def _bd8(blk16):
    return jnp.kron(jnp.eye(8, dtype=jnp.float32), blk16)


def _pad16(m):
    z = jnp.zeros((16, 16), jnp.float32)
    return lax.dynamic_update_slice(z, m, (0, 0))


def kernel(x, edge_index, batch, W1_l, W1_r, b1, W2_l, W2_r, b2, W_out, b_out):
    n = x.shape[0]
    e = edge_index.shape[1]
    f_in = x.shape[1]
    n_pad = -(-n // (NS * RW)) * (NS * RW)
    src = edge_index[0].astype(jnp.int32)
    dst = edge_index[1].astype(jnp.int32)
    x3 = x.reshape(n // 8, 8, f_in)
    xa3 = jnp.concatenate(
        [x3, jnp.ones((n // 8, 8, 1), x.dtype),
         jnp.zeros((n // 8, 8, 15 - f_in), x.dtype)], axis=2)
    xa_p = jnp.pad(xa3.reshape(n // 8, 128), ((0, (n_pad - n) // 8), (0, 0)))
    xa = xa_p.reshape(n_pad, 16)

    p = _sc_agg_layer1(n_pad, e)(xa, src, dst)
    p_p = p.reshape(NC, n_pad // 8, 128)

    w1l = W1_l.T
    w1r = W1_r.T
    w2r = W2_r.T
    w2l = W2_l.T
    wo = W_out.T
    ri = jnp.arange(128)[:, None]
    cj = jnp.arange(128)[None, :]
    b5 = (ri == (cj // 16) * 16 + f_in).astype(jnp.float32)
    bmax = (ri == (cj // 16) * 16).astype(jnp.float32)
    s5 = ((ri // 16 == cj // 16) & (ri % 16 < wo.shape[1])).astype(
        jnp.float32)
    shs = [(ri == cj + i).astype(jnp.float32) for i in range(1, 5)]

    h1lo_p, h1hi_p = _dense1(
        p_p, xa_p,
        _bd8(_pad16(w1l[:, :16])), _bd8(_pad16(w1l[:, 16:])),
        _bd8(_pad16(w1r[:, :16])), _bd8(_pad16(w1r[:, 16:])),
        jnp.tile(b1[:16], 8)[None, :], jnp.tile(b1[16:], 8)[None, :],
        b5)
    r2lo_p, r2hi_p, degb_p = _dense_r2(
        h1lo_p, h1hi_p, p_p,
        _bd8(w2r[:16, :16]), _bd8(w2r[16:, :16]),
        _bd8(w2r[:16, 16:]), _bd8(w2r[16:, 16:]),
        jnp.tile(b2[:16], 8)[None, :], jnp.tile(b2[16:], 8)[None, :],
        b5)

    q = _sc_agg_layer2(n_pad, e)(
        h1lo_p.reshape(n_pad, 16), h1hi_p.reshape(n_pad, 16), src, dst)
    q_p = q.reshape(NC, n_pad // 8, 128)

    bo_t = jnp.tile(jnp.pad(b_out, (0, 16 - wo.shape[1])), 8)[None, :]
    ra = jnp.arange(PB2)[None, :, None]
    ma = jnp.arange(8 * PB2)[None, None, :]
    aa = jnp.arange(8)[:, None, None]
    g = (ma == 8 * ra + aa).astype(jnp.float32)
    out_t = _dense2(
        q_p, r2lo_p, r2hi_p, degb_p,
        _bd8(w2l[:16, :16]), _bd8(w2l[16:, :16]),
        _bd8(w2l[:16, 16:]), _bd8(w2l[16:, 16:]),
        _bd8(_pad16(wo[:16, :])), _bd8(_pad16(wo[16:, :])),
        bo_t, shs[0], shs[1], shs[2], shs[3], bmax, s5, g)

    return out_t[:, :n].T


# speedup vs baseline: 1.5083x; 1.5083x over previous
import functools

import jax
import jax.numpy as jnp
from jax import lax
from jax.experimental import pallas as pl
from jax.experimental.pallas import tpu as pltpu
from jax.experimental.pallas import tpu_sc as plsc

NC = 2
NS = 16
EK = 744
RW = 400
PB = 400


def _zero_rows(rows_v, nrows):
    def zrow(i, carry):
        rows_v[0, i, :] = jnp.zeros((16,), jnp.float32)
        return carry
    lax.fori_loop(0, nrows, zrow, 0)


def _zero_agg(agg, rows_v, rbase, rpt):
    def zcp(j, carry):
        pltpu.sync_copy(rows_v.at[0, pl.ds(0, RW)],
                        agg.at[pl.ds(rbase + j * RW, RW)])
        return carry
    lax.fori_loop(0, rpt // RW, zcp, 0)


RING = 3


def _agg_edges(table, src_a, dst_a, agg, src_v, dst_v, rows_v,
               lsem, gsem, ssems, ebase, nch, tail):
    def start_load(j, b3):
        off = ebase + j * EK
        pltpu.async_copy(src_a.at[pl.ds(off, EK)], src_v.at[b3], lsem)
        pltpu.async_copy(dst_a.at[pl.ds(off, EK)], dst_v.at[b3], lsem)

    def wait_load(b3):
        pltpu.make_async_copy(src_a.at[pl.ds(0, EK)], src_v.at[b3],
                              lsem).wait()
        pltpu.make_async_copy(dst_a.at[pl.ds(0, EK)], dst_v.at[b3],
                              lsem).wait()

    def wait_scatter(b2, b3):
        pltpu.make_async_copy(rows_v.at[b2], agg.at[dst_v.at[b3]],
                              ssems[b2]).wait()

    start_load(0, 0)
    n_outer = (nch + 5) // 6

    def outer(t, carry):
        j0 = t * 6
        for k in range(6):
            j = j0 + k
            b3 = k % 3
            b2 = k % 2

            @pl.when(j < nch)
            def _():
                wait_load(b3)

                @pl.when(j >= 2)
                def _():
                    wait_scatter(b2, (k + 1) % 3)

                @pl.when(j + 1 < nch)
                def _():
                    start_load(j + 1, (k + 1) % 3)

                pltpu.async_copy(table.at[src_v.at[b3]], rows_v.at[b2],
                                 gsem).wait()
                pltpu.async_copy(rows_v.at[b2], agg.at[dst_v.at[b3]],
                                 ssems[b2], add=True)
        return carry

    lax.fori_loop(0, n_outer, outer, 0)
    for k in (2, 1):
        if nch - k >= 0:
            wait_scatter((nch - k) % 2, (nch - k) % 3)
    if tail:
        off = ebase + nch * EK
        pltpu.sync_copy(src_a.at[pl.ds(off, tail)],
                        src_v.at[0, pl.ds(0, tail)])
        pltpu.sync_copy(dst_a.at[pl.ds(off, tail)],
                        dst_v.at[0, pl.ds(0, tail)])
        pltpu.async_copy(table.at[src_v.at[0, pl.ds(0, tail)]],
                         rows_v.at[0, pl.ds(0, tail)], gsem).wait()
        pltpu.sync_copy(rows_v.at[0, pl.ds(0, tail)],
                        agg.at[dst_v.at[0, pl.ds(0, tail)]], add=True)


def _write_out(agg, rows_v, out, c, rbase, rpt):
    def wcp(j, carry):
        r0 = rbase + j * RW
        pltpu.sync_copy(agg.at[pl.ds(r0, RW)], rows_v.at[0, pl.ds(0, RW)])
        pltpu.sync_copy(rows_v.at[0, pl.ds(0, RW)], out.at[c, pl.ds(r0, RW)])
        return carry
    lax.fori_loop(0, rpt // RW, wcp, 0)


def _sc_agg_layer1(n, e):
    mesh = plsc.VectorSubcoreMesh(core_axis_name="c", subcore_axis_name="s")

    @functools.partial(
        pl.kernel,
        out_type=jax.ShapeDtypeStruct((NC, n, 16), jnp.float32),
        mesh=mesh,
        scratch_types=[
            pltpu.VMEM_SHARED((n, 16), jnp.float32),
            pltpu.VMEM((RING, EK), jnp.int32),
            pltpu.VMEM((RING, EK), jnp.int32),
            pltpu.VMEM((2, EK, 16), jnp.float32),
            pltpu.SemaphoreType.DMA,
            pltpu.SemaphoreType.DMA,
            pltpu.SemaphoreType.DMA,
            pltpu.SemaphoreType.DMA,
        ],
        compiler_params=pltpu.CompilerParams(use_tc_tiling_on_sc=False),
    )
    def k(table, src_a, dst_a, out, agg, src_v, dst_v, rows_v,
          lsem, gsem, ssem0, ssem1):
        c = lax.axis_index("c")
        s = lax.axis_index("s")
        rpt = n // NS
        rbase = s * rpt
        _zero_rows(rows_v, RW)
        _zero_agg(agg, rows_v, rbase, rpt)
        plsc.subcore_barrier()
        wid = s * NC + c
        ept = e // (NC * NS)
        _agg_edges(table, src_a, dst_a, agg, src_v, dst_v, rows_v,
                   lsem, gsem, (ssem0, ssem1),
                   wid * ept, ept // EK, ept % EK)
        plsc.subcore_barrier()
        _write_out(agg, rows_v, out, c, rbase, rpt)

    return k


def _sc_agg_layer2(n, e):
    mesh = plsc.VectorSubcoreMesh(core_axis_name="c", subcore_axis_name="s")

    @functools.partial(
        pl.kernel,
        out_type=jax.ShapeDtypeStruct((NC, n, 16), jnp.float32),
        mesh=mesh,
        scratch_types=[
            pltpu.VMEM_SHARED((n, 16), jnp.float32),
            pltpu.VMEM((RING, EK), jnp.int32),
            pltpu.VMEM((RING, EK), jnp.int32),
            pltpu.VMEM((2, EK, 16), jnp.float32),
            pltpu.SemaphoreType.DMA,
            pltpu.SemaphoreType.DMA,
            pltpu.SemaphoreType.DMA,
            pltpu.SemaphoreType.DMA,
        ],
        compiler_params=pltpu.CompilerParams(use_tc_tiling_on_sc=False),
    )
    def k(t_lo, t_hi, src_a, dst_a, out, agg, src_v, dst_v, rows_v,
          lsem, gsem, ssem0, ssem1):
        c = lax.axis_index("c")
        s = lax.axis_index("s")
        rpt = n // NS
        rbase = s * rpt
        _zero_rows(rows_v, RW)
        _zero_agg(agg, rows_v, rbase, rpt)
        plsc.subcore_barrier()
        ept = e // NS
        ebase = s * ept
        ssems = (ssem0, ssem1)

        @pl.when(c == 0)
        def _():
            _agg_edges(t_lo, src_a, dst_a, agg, src_v, dst_v, rows_v,
                       lsem, gsem, ssems, ebase, ept // EK, ept % EK)

        @pl.when(c == 1)
        def _():
            _agg_edges(t_hi, src_a, dst_a, agg, src_v, dst_v, rows_v,
                       lsem, gsem, ssems, ebase, ept // EK, ept % EK)

        plsc.subcore_barrier()
        _write_out(agg, rows_v, out, c, rbase, rpt)

    return k


def _dense1(p_p, xa_p, w1l_lo, w1l_hi, w1r_lo, w1r_hi, b1lo, b1hi, b5):
    m = p_p.shape[1]

    def body(p_ref, xa_ref, a_ref, b_ref, c_ref, d_ref, e_ref, f_ref,
             b5_ref, lo_ref, hi_ref):
        dot = functools.partial(jnp.dot, preferred_element_type=jnp.float32)
        psum = p_ref[0] + p_ref[1]
        degb = jnp.maximum(dot(psum, b5_ref[...]), 1.0)
        mean = psum / degb
        xa = xa_ref[...]
        lo_ref[...] = jnp.maximum(
            dot(mean, a_ref[...]) + dot(xa, c_ref[...]) + e_ref[...], 0.0)
        hi_ref[...] = jnp.maximum(
            dot(mean, b_ref[...]) + dot(xa, d_ref[...]) + f_ref[...], 0.0)

    full = pl.BlockSpec((128, 128), lambda i: (0, 0))
    row = pl.BlockSpec((1, 128), lambda i: (0, 0))
    blk = pl.BlockSpec((PB, 128), lambda i: (i, 0))
    return pl.pallas_call(
        body,
        grid=(m // PB,),
        in_specs=[pl.BlockSpec((NC, PB, 128), lambda i: (0, i, 0)), blk,
                  full, full, full, full, row, row, full],
        out_specs=[blk, blk],
        out_shape=[jax.ShapeDtypeStruct((m, 128), jnp.float32)] * 2,
    )(p_p, xa_p, w1l_lo, w1l_hi, w1r_lo, w1r_hi, b1lo, b1hi, b5)


def _dense_r2(h1lo_p, h1hi_p, p_p, w2r_ll, w2r_hl, w2r_lh, w2r_hh,
              b2lo, b2hi, b5):
    m = p_p.shape[1]

    def body(lo_ref, hi_ref, p_ref, g_ref, h_ref, i_ref, j_ref,
             k_ref, l_ref, b5_ref, r2lo_ref, r2hi_ref, deg_ref):
        dot = functools.partial(jnp.dot, preferred_element_type=jnp.float32)
        h1lo = lo_ref[...]
        h1hi = hi_ref[...]
        r2lo_ref[...] = dot(h1lo, g_ref[...]) + dot(h1hi, h_ref[...]) \
            + k_ref[...]
        r2hi_ref[...] = dot(h1lo, i_ref[...]) + dot(h1hi, j_ref[...]) \
            + l_ref[...]
        psum = p_ref[0] + p_ref[1]
        deg_ref[...] = jnp.maximum(dot(psum, b5_ref[...]), 1.0)

    full = pl.BlockSpec((128, 128), lambda i: (0, 0))
    row = pl.BlockSpec((1, 128), lambda i: (0, 0))
    blk = pl.BlockSpec((PB, 128), lambda i: (i, 0))
    return pl.pallas_call(
        body,
        grid=(m // PB,),
        in_specs=[blk, blk, pl.BlockSpec((NC, PB, 128), lambda i: (0, i, 0)),
                  full, full, full, full, row, row, full],
        out_specs=[blk, blk, blk],
        out_shape=[jax.ShapeDtypeStruct((m, 128), jnp.float32)] * 3,
    )(h1lo_p, h1hi_p, p_p, w2r_ll, w2r_hl, w2r_lh, w2r_hh, b2lo, b2hi, b5)


PB2 = 128


def _dense2(q_p, r2lo_p, r2hi_p, degb_p, w2l_ll, w2l_hl, w2l_lh, w2l_hh,
            wo_lo, wo_hi, bo, sh1, sh2, sh3, sh4, bmax, s5, g):
    m = q_p.shape[1]

    def body(q_ref, r2lo_ref, r2hi_ref, deg_ref, a_ref, b_ref, c_ref, d_ref,
             e_ref, f_ref, bo_ref, s1_ref, s2_ref, s3_ref, s4_ref,
             bm_ref, s5_ref, g_ref, out_ref):
        dot = functools.partial(jnp.dot, preferred_element_type=jnp.float32)
        degb = deg_ref[...]
        mlo = q_ref[0] / degb
        mhi = q_ref[1] / degb
        h2lo = jnp.maximum(
            dot(mlo, a_ref[...]) + dot(mhi, b_ref[...]) + r2lo_ref[...], 0.0)
        h2hi = jnp.maximum(
            dot(mlo, c_ref[...]) + dot(mhi, d_ref[...]) + r2hi_ref[...], 0.0)
        lg = dot(h2lo, e_ref[...]) + dot(h2hi, f_ref[...]) + bo_ref[...]
        mx = jnp.maximum(lg, dot(lg, s1_ref[...]))
        mx = jnp.maximum(mx, dot(lg, s2_ref[...]))
        mx = jnp.maximum(mx, dot(lg, s3_ref[...]))
        mx = jnp.maximum(mx, dot(lg, s4_ref[...]))
        mb = dot(mx, bm_ref[...])
        ez = jnp.exp(lg - mb)
        sb = dot(ez, s5_ref[...])
        probs = ez / sb
        pt = jnp.transpose(probs)
        acc = dot(pt[0:5, :], g_ref[0])
        for a in range(1, 8):
            acc = acc + dot(pt[16 * a:16 * a + 5, :], g_ref[a])
        out_ref[...] = acc

    full = pl.BlockSpec((128, 128), lambda i: (0, 0))
    row = pl.BlockSpec((1, 128), lambda i: (0, 0))
    blk = pl.BlockSpec((PB2, 128), lambda i: (i, 0))
    return pl.pallas_call(
        body,
        grid=(m // PB2,),
        in_specs=[pl.BlockSpec((NC, PB2, 128), lambda i: (0, i, 0)),
                  blk, blk, blk,
                  full, full, full, full, full, full, row,
                  full, full, full, full, full, full,
                  pl.BlockSpec((8, PB2, 8 * PB2), lambda i: (0, 0, 0))],
        out_specs=pl.BlockSpec((5, 8 * PB2), lambda i: (0, i)),
        out_shape=jax.ShapeDtypeStruct((5, 8 * m), jnp.float32),
    )(q_p, r2lo_p, r2hi_p, degb_p, w2l_ll, w2l_hl, w2l_lh, w2l_hh,
      wo_lo, wo_hi, bo, sh1, sh2, sh3, sh4, bmax, s5, g)


def _bd8(blk16):
    return jnp.kron(jnp.eye(8, dtype=jnp.float32), blk16)


def _pad16(m):
    z = jnp.zeros((16, 16), jnp.float32)
    return lax.dynamic_update_slice(z, m, (0, 0))


def kernel(x, edge_index, batch, W1_l, W1_r, b1, W2_l, W2_r, b2, W_out, b_out):
    n = x.shape[0]
    e = edge_index.shape[1]
    f_in = x.shape[1]
    n_pad = -(-n // (NS * RW)) * (NS * RW)
    src = edge_index[0].astype(jnp.int32)
    dst = edge_index[1].astype(jnp.int32)
    x3 = x.reshape(n // 8, 8, f_in)
    xa3 = jnp.concatenate(
        [x3, jnp.ones((n // 8, 8, 1), x.dtype),
         jnp.zeros((n // 8, 8, 15 - f_in), x.dtype)], axis=2)
    xa_p = jnp.pad(xa3.reshape(n // 8, 128), ((0, (n_pad - n) // 8), (0, 0)))
    xa = xa_p.reshape(n_pad, 16)

    p = _sc_agg_layer1(n_pad, e)(xa, src, dst)
    p_p = p.reshape(NC, n_pad // 8, 128)

    w1l = W1_l.T
    w1r = W1_r.T
    w2r = W2_r.T
    w2l = W2_l.T
    wo = W_out.T
    ri = jnp.arange(128)[:, None]
    cj = jnp.arange(128)[None, :]
    b5 = (ri == (cj // 16) * 16 + f_in).astype(jnp.float32)
    bmax = (ri == (cj // 16) * 16).astype(jnp.float32)
    s5 = ((ri // 16 == cj // 16) & (ri % 16 < wo.shape[1])).astype(
        jnp.float32)
    shs = [(ri == cj + i).astype(jnp.float32) for i in range(1, 5)]

    h1lo_p, h1hi_p = _dense1(
        p_p, xa_p,
        _bd8(_pad16(w1l[:, :16])), _bd8(_pad16(w1l[:, 16:])),
        _bd8(_pad16(w1r[:, :16])), _bd8(_pad16(w1r[:, 16:])),
        jnp.tile(b1[:16], 8)[None, :], jnp.tile(b1[16:], 8)[None, :],
        b5)
    r2lo_p, r2hi_p, degb_p = _dense_r2(
        h1lo_p, h1hi_p, p_p,
        _bd8(w2r[:16, :16]), _bd8(w2r[16:, :16]),
        _bd8(w2r[:16, 16:]), _bd8(w2r[16:, 16:]),
        jnp.tile(b2[:16], 8)[None, :], jnp.tile(b2[16:], 8)[None, :],
        b5)

    q = _sc_agg_layer2(n_pad, e)(
        h1lo_p.reshape(n_pad, 16), h1hi_p.reshape(n_pad, 16), src, dst)
    q_p = q.reshape(NC, n_pad // 8, 128)

    bo_t = jnp.tile(jnp.pad(b_out, (0, 16 - wo.shape[1])), 8)[None, :]
    ra = jnp.arange(PB2)[None, :, None]
    ma = jnp.arange(8 * PB2)[None, None, :]
    aa = jnp.arange(8)[:, None, None]
    g = (ma == 8 * ra + aa).astype(jnp.float32)
    out_t = _dense2(
        q_p, r2lo_p, r2hi_p, degb_p,
        _bd8(w2l[:16, :16]), _bd8(w2l[16:, :16]),
        _bd8(w2l[:16, 16:]), _bd8(w2l[16:, 16:]),
        _bd8(_pad16(wo[:16, :])), _bd8(_pad16(wo[16:, :])),
        bo_t, shs[0], shs[1], shs[2], shs[3], bmax, s5, g)

    return out_t[:, :n].T
